# baseline (speedup 1.0000x reference)
import functools

import numpy as np

import jax
import jax.numpy as jnp
from jax import lax
from jax.experimental import pallas as pl
from jax.experimental.pallas import tpu as pltpu
from jax.experimental.pallas import tpu_sc as plsc

VOCAB = 8192
DIM = 256
N_TOK = 4096
COMMIT = 0.25

TR = 1024
TV = 2048
RT = N_TOK // TR
VT = VOCAB // TV
BPR = TR // 256

_COLSF = np.arange(TV, dtype=np.float32).reshape(1, TV)
_COLSI = np.arange(TV, dtype=np.int32).reshape(1, TV)
_IDENT = np.eye(256, dtype=np.float32)


def _fused_body(x_ref, c_ref, colsf_ref, colsi_ref,
                enc_ref, mini_ref, loss_ref, perp_ref,
                xm2_ref, xsq_ref, csq_ref, cnt_ref,
                minv_ref, mini_c_ref, mini0_ref, mini1_ref,
                oh0_ref, oh1_ref, acc_ref, sem0, sem1):
    r = pl.program_id(0)
    v = pl.program_id(1)

    @pl.when(r < RT)
    def _argmin_phase():
        @pl.when(v == 0)
        def _():
            xb = x_ref[...]
            xm2_ref[...] = xb * (-2.0)
            xsq_ref[...] = jnp.sum(xb * xb, axis=1, keepdims=True)

        c = c_ref[...]

        @pl.when(r == 0)
        def _():
            ones = jnp.ones((1, DIM), jnp.float32)
            csq_ref[0:1, pl.ds(v * TV, TV)] = lax.dot_general(
                ones, c * c, (((1,), (1,)), ((), ())),
                preferred_element_type=jnp.float32)

        mm = lax.dot_general(xm2_ref[...], c, (((1,), (1,)), ((), ())),
                             preferred_element_type=jnp.float32)
        t = xsq_ref[...] + csq_ref[0:1, pl.ds(v * TV, TV)]
        d = t + mm
        lmin = jnp.min(d, axis=1, keepdims=True)
        largf = jnp.min(jnp.where(d == lmin, colsf_ref[...], 1e9),
                        axis=1, keepdims=True)
        larg = largf.astype(jnp.int32) + v * TV

        @pl.when(v == 0)
        def _():
            minv_ref[...] = lmin
            mini_c_ref[...] = larg

        @pl.when(v > 0)
        def _():
            better = lmin < minv_ref[...]
            minv_ref[...] = jnp.where(better, lmin, minv_ref[...])
            mini_c_ref[...] = jnp.where(better, larg, mini_c_ref[...])

        @pl.when(v == VT - 1)
        def _():
            mini_ref[...] = mini_c_ref[...].reshape(1, TR, 1)

            @pl.when(r % 2 == 0)
            def _():
                mini0_ref[...] = mini_c_ref[...]

            @pl.when(r % 2 == 1)
            def _():
                mini1_ref[...] = mini_c_ref[...]

            s_ = jnp.sum(minv_ref[...])

            @pl.when(r == 0)
            def _():
                acc_ref[0, 0] = s_

            @pl.when(r > 0)
            def _():
                acc_ref[0, 0] = acc_ref[0, 0] + s_

    @pl.when(r >= 1)
    def _onehot_phase():
        idxp = jnp.where((r - 1) % 2 == 0, mini0_ref[...], mini1_ref[...])
        idxs = idxp - v * TV
        u = (r - 1) * VT + v
        dst = enc_ref.at[pl.ds((r - 1) * TR, TR), pl.ds(v * TV, TV)]

        def _emit(oh_ref, sem):
            @pl.when(u >= 2)
            def _():
                pltpu.make_async_copy(oh_ref, dst, sem).wait()
            oh_ref[...] = (idxs == colsi_ref[...]).astype(jnp.float32)
            colsum = jnp.sum(oh_ref[...], axis=0, keepdims=True)

            @pl.when(r == 1)
            def _():
                cnt_ref[0:1, pl.ds(v * TV, TV)] = colsum

            @pl.when(r > 1)
            def _():
                cnt_ref[0:1, pl.ds(v * TV, TV)] = (
                    cnt_ref[0:1, pl.ds(v * TV, TV)] + colsum)

            pltpu.make_async_copy(oh_ref, dst, sem).start()

        @pl.when(u % 2 == 0)
        def _():
            _emit(oh0_ref, sem0)

        @pl.when(u % 2 == 1)
        def _():
            _emit(oh1_ref, sem1)

    @pl.when((r == RT) & (v == VT - 1))
    def _final():
        drain = enc_ref.at[pl.ds(0, TR), pl.ds(0, TV)]
        pltpu.make_async_copy(oh0_ref, drain, sem0).wait()
        pltpu.make_async_copy(oh1_ref, drain, sem1).wait()
        m = acc_ref[0, 0] * (1.0 / (N_TOK * DIM))
        loss_ref[0, 0] = m + COMMIT * m
        p = cnt_ref[...] * (1.0 / N_TOK)
        perp_ref[0, 0] = jnp.exp(-jnp.sum(p * jnp.log(p + 1e-10)))


_fused_call = pl.pallas_call(
    _fused_body,
    grid=(RT + 1, VT),
    in_specs=[
        pl.BlockSpec((TR, DIM), lambda r, v: (jnp.minimum(r, RT - 1), 0)),
        pl.BlockSpec((TV, DIM), lambda r, v: (jnp.where(r == RT, VT - 1, v), 0)),
        pl.BlockSpec((1, TV), lambda r, v: (0, 0)),
        pl.BlockSpec((1, TV), lambda r, v: (0, 0)),
    ],
    out_specs=[
        pl.BlockSpec(memory_space=pl.ANY),
        pl.BlockSpec((1, TR, 1), lambda r, v: (jnp.minimum(r, RT - 1), 0, 0)),
        pl.BlockSpec(memory_space=pltpu.SMEM),
        pl.BlockSpec(memory_space=pltpu.SMEM),
    ],
    out_shape=[
        jax.ShapeDtypeStruct((N_TOK, VOCAB), jnp.float32),
        jax.ShapeDtypeStruct((RT, TR, 1), jnp.int32),
        jax.ShapeDtypeStruct((1, 1), jnp.float32),
        jax.ShapeDtypeStruct((1, 1), jnp.float32),
    ],
    scratch_shapes=[
        pltpu.VMEM((TR, DIM), jnp.float32),
        pltpu.VMEM((TR, 1), jnp.float32),
        pltpu.VMEM((1, VOCAB), jnp.float32),
        pltpu.VMEM((1, VOCAB), jnp.float32),
        pltpu.VMEM((TR, 1), jnp.float32),
        pltpu.VMEM((TR, 1), jnp.int32),
        pltpu.VMEM((TR, 1), jnp.int32),
        pltpu.VMEM((TR, 1), jnp.int32),
        pltpu.VMEM((TR, TV), jnp.float32),
        pltpu.VMEM((TR, TV), jnp.float32),
        pltpu.SMEM((1, 1), jnp.float32),
        pltpu.SemaphoreType.DMA,
        pltpu.SemaphoreType.DMA,
    ],
)


_NC = 2
_NS = 16
_NW = _NC * _NS
_BPW = N_TOK // _NW


@functools.cache
def _make_sc_gather():
    @functools.partial(
        pl.kernel,
        mesh=plsc.VectorSubcoreMesh(core_axis_name="c", subcore_axis_name="s"),
        out_type=jax.ShapeDtypeStruct((N_TOK, DIM), jnp.float32),
        scratch_types=[
            pltpu.VMEM((_BPW,), jnp.int32),
            pltpu.VMEM((_BPW, DIM), jnp.float32),
            pltpu.SemaphoreType.DMA,
        ],
    )
    def _sc_gather(table_hbm, idx_hbm, out_hbm, idx_v, rows_v, sem):
        wid = lax.axis_index("s") * _NC + lax.axis_index("c")
        base = wid * _BPW
        pltpu.sync_copy(idx_hbm.at[pl.ds(base, _BPW)], idx_v)
        pltpu.async_copy(table_hbm.at[idx_v], rows_v, sem).wait()
        pltpu.sync_copy(rows_v, out_hbm.at[pl.ds(base, _BPW)])

    return _sc_gather


_IDENT = np.eye(256, dtype=np.float32)


def _qst_body(x_ref, q_ref, ident_ref, qst_ref):
    x = x_ref[0]
    qt = lax.dot_general(q_ref[...], ident_ref[...], (((0,), (0,)), ((), ())),
                         preferred_element_type=jnp.float32)
    qst_ref[0] = x + (qt - x)


_qst_call = pl.pallas_call(
    _qst_body,
    grid=(16,),
    in_specs=[
        pl.BlockSpec((1, DIM, 256), lambda b: (b, 0, 0)),
        pl.BlockSpec((256, DIM), lambda b: (b, 0)),
        pl.BlockSpec((256, 256), lambda b: (0, 0)),
    ],
    out_specs=pl.BlockSpec((1, DIM, 256), lambda b: (b, 0, 0)),
    out_shape=jax.ShapeDtypeStruct((16, DIM, 256), jnp.float32),
)


def kernel(inputs, codebook):
    B, C, H, W = inputs.shape
    x = jnp.transpose(inputs, (0, 2, 3, 1)).reshape(N_TOK, DIM)
    enc, mini, loss, perp = _fused_call(x, codebook, _COLSF, _COLSI)
    idx_flat = mini.reshape(N_TOK)
    q = _make_sc_gather()(codebook, idx_flat)
    qst = _qst_call(inputs.reshape(B, C, H * W), q, _IDENT)
    qst_out = qst.reshape(B, C, H, W)
    return (loss.reshape(()), qst_out, perp.reshape(()), enc)


# speedup vs baseline: 1.1293x; 1.1293x over previous
import functools

import numpy as np

import jax
import jax.numpy as jnp
from jax import lax
from jax.experimental import pallas as pl
from jax.experimental.pallas import tpu as pltpu
from jax.experimental.pallas import tpu_sc as plsc

VOCAB = 8192
DIM = 256
N_TOK = 4096
COMMIT = 0.25

TR = 1024
TV = 2048
RT = N_TOK // TR
VT = VOCAB // TV
BPR = TR // 256

_COLSF = np.arange(TV, dtype=np.float32).reshape(1, TV)
_COLSI = np.arange(TV, dtype=np.int32).reshape(1, TV)
_IDENT = np.eye(256, dtype=np.float32)


def _fused_body(x_ref, c_ref, colsf_ref, colsi_ref,
                enc_ref, mini_ref, loss_ref, perp_ref,
                xm2_ref, xsq_ref, csq_ref, cnt_ref,
                minv_ref, mini_c_ref, mini0_ref, mini1_ref,
                oh0_ref, oh1_ref, acc_ref, sem0, sem1):
    r = pl.program_id(0)
    v = pl.program_id(1)

    @pl.when(r < RT)
    def _argmin_phase():
        @pl.when(v == 0)
        def _():
            xb = x_ref[...]
            xm2_ref[...] = xb * (-2.0)
            xsq_ref[...] = jnp.sum(xb * xb, axis=1, keepdims=True)

        c = c_ref[...]

        @pl.when(r == 0)
        def _():
            ones = jnp.ones((1, DIM), jnp.float32)
            csq_ref[0:1, pl.ds(v * TV, TV)] = lax.dot_general(
                ones, c * c, (((1,), (1,)), ((), ())),
                preferred_element_type=jnp.float32)

        mm = lax.dot_general(xm2_ref[...], c, (((1,), (1,)), ((), ())),
                             preferred_element_type=jnp.float32)
        t = xsq_ref[...] + csq_ref[0:1, pl.ds(v * TV, TV)]
        d = t + mm
        lmin = jnp.min(d, axis=1, keepdims=True)
        largf = jnp.min(jnp.where(d == lmin, colsf_ref[...], 1e9),
                        axis=1, keepdims=True)
        larg = largf.astype(jnp.int32) + v * TV

        @pl.when(v == 0)
        def _():
            minv_ref[...] = lmin
            mini_c_ref[...] = larg

        @pl.when(v > 0)
        def _():
            better = lmin < minv_ref[...]
            minv_ref[...] = jnp.where(better, lmin, minv_ref[...])
            mini_c_ref[...] = jnp.where(better, larg, mini_c_ref[...])

        @pl.when(v == VT - 1)
        def _():
            mini_ref[...] = mini_c_ref[...].reshape(1, TR, 1)

            @pl.when(r % 2 == 0)
            def _():
                mini0_ref[...] = mini_c_ref[...]

            @pl.when(r % 2 == 1)
            def _():
                mini1_ref[...] = mini_c_ref[...]

            s_ = jnp.sum(minv_ref[...])

            @pl.when(r == 0)
            def _():
                acc_ref[0, 0] = s_

            @pl.when(r > 0)
            def _():
                acc_ref[0, 0] = acc_ref[0, 0] + s_

    @pl.when(r >= 1)
    def _onehot_phase():
        idxp = jnp.where((r - 1) % 2 == 0, mini0_ref[...], mini1_ref[...])
        idxs = idxp - v * TV
        u = (r - 1) * VT + v
        dst = enc_ref.at[pl.ds((r - 1) * TR, TR), pl.ds(v * TV, TV)]

        def _emit(oh_ref, sem):
            @pl.when(u >= 2)
            def _():
                pltpu.make_async_copy(oh_ref, dst, sem).wait()
            oh_ref[...] = (idxs == colsi_ref[...]).astype(jnp.float32)
            colsum = jnp.sum(oh_ref[...], axis=0, keepdims=True)

            @pl.when(r == 1)
            def _():
                cnt_ref[0:1, pl.ds(v * TV, TV)] = colsum

            @pl.when(r > 1)
            def _():
                cnt_ref[0:1, pl.ds(v * TV, TV)] = (
                    cnt_ref[0:1, pl.ds(v * TV, TV)] + colsum)

            pltpu.make_async_copy(oh_ref, dst, sem).start()

        @pl.when(u % 2 == 0)
        def _():
            _emit(oh0_ref, sem0)

        @pl.when(u % 2 == 1)
        def _():
            _emit(oh1_ref, sem1)

    @pl.when((r == RT) & (v == VT - 1))
    def _final():
        drain = enc_ref.at[pl.ds(0, TR), pl.ds(0, TV)]
        pltpu.make_async_copy(oh0_ref, drain, sem0).wait()
        pltpu.make_async_copy(oh1_ref, drain, sem1).wait()
        m = acc_ref[0, 0] * (1.0 / (N_TOK * DIM))
        loss_ref[0, 0] = m + COMMIT * m
        p = cnt_ref[...] * (1.0 / N_TOK)
        perp_ref[0, 0] = jnp.exp(-jnp.sum(p * jnp.log(p + 1e-10)))


_fused_call = pl.pallas_call(
    _fused_body,
    grid=(RT + 1, VT),
    in_specs=[
        pl.BlockSpec((TR, DIM), lambda r, v: (jnp.minimum(r, RT - 1), 0)),
        pl.BlockSpec((TV, DIM), lambda r, v: (jnp.where(r == RT, VT - 1, v), 0)),
        pl.BlockSpec((1, TV), lambda r, v: (0, 0)),
        pl.BlockSpec((1, TV), lambda r, v: (0, 0)),
    ],
    out_specs=[
        pl.BlockSpec(memory_space=pl.ANY),
        pl.BlockSpec((1, TR, 1), lambda r, v: (jnp.minimum(r, RT - 1), 0, 0)),
        pl.BlockSpec(memory_space=pltpu.SMEM),
        pl.BlockSpec(memory_space=pltpu.SMEM),
    ],
    out_shape=[
        jax.ShapeDtypeStruct((N_TOK, VOCAB), jnp.float32),
        jax.ShapeDtypeStruct((RT, TR, 1), jnp.int32),
        jax.ShapeDtypeStruct((1, 1), jnp.float32),
        jax.ShapeDtypeStruct((1, 1), jnp.float32),
    ],
    scratch_shapes=[
        pltpu.VMEM((TR, DIM), jnp.float32),
        pltpu.VMEM((TR, 1), jnp.float32),
        pltpu.VMEM((1, VOCAB), jnp.float32),
        pltpu.VMEM((1, VOCAB), jnp.float32),
        pltpu.VMEM((TR, 1), jnp.float32),
        pltpu.VMEM((TR, 1), jnp.int32),
        pltpu.VMEM((TR, 1), jnp.int32),
        pltpu.VMEM((TR, 1), jnp.int32),
        pltpu.VMEM((TR, TV), jnp.float32),
        pltpu.VMEM((TR, TV), jnp.float32),
        pltpu.SMEM((1, 1), jnp.float32),
        pltpu.SemaphoreType.DMA,
        pltpu.SemaphoreType.DMA,
    ],
)


_NC = 2
_NS = 16
_NW = _NC * _NS
_BPW = N_TOK // _NW


@functools.cache
def _make_sc_gather():
    @functools.partial(
        pl.kernel,
        mesh=plsc.VectorSubcoreMesh(core_axis_name="c", subcore_axis_name="s"),
        out_type=jax.ShapeDtypeStruct((N_TOK, DIM), jnp.float32),
        scratch_types=[
            pltpu.VMEM((_BPW,), jnp.int32),
            pltpu.VMEM((_BPW, DIM), jnp.float32),
            pltpu.SemaphoreType.DMA,
        ],
    )
    def _sc_gather(table_hbm, idx_hbm, out_hbm, idx_v, rows_v, sem):
        wid = lax.axis_index("s") * _NC + lax.axis_index("c")
        base = wid * _BPW
        pltpu.sync_copy(idx_hbm.at[pl.ds(base, _BPW)], idx_v)
        pltpu.async_copy(table_hbm.at[idx_v], rows_v, sem).wait()
        pltpu.sync_copy(rows_v, out_hbm.at[pl.ds(base, _BPW)])

    return _sc_gather


def _qst_body(x_ref, q_ref, qst_ref):
    x = x_ref[...]
    q = q_ref[...]
    qst_ref[...] = x + (q - x)


_qst_call = pl.pallas_call(
    _qst_body,
    grid=(RT,),
    in_specs=[
        pl.BlockSpec((TR, DIM), lambda r: (r, 0)),
        pl.BlockSpec((TR, DIM), lambda r: (r, 0)),
    ],
    out_specs=pl.BlockSpec((TR, DIM), lambda r: (r, 0)),
    out_shape=jax.ShapeDtypeStruct((N_TOK, DIM), jnp.float32),
)


def kernel(inputs, codebook):
    B, C, H, W = inputs.shape
    x = jnp.transpose(inputs, (0, 2, 3, 1)).reshape(N_TOK, DIM)
    enc, mini, loss, perp = _fused_call(x, codebook, _COLSF, _COLSI)
    idx_flat = mini.reshape(N_TOK)
    q = _make_sc_gather()(codebook, idx_flat)
    qst = _qst_call(x, q)
    qst_out = jnp.transpose(qst.reshape(B, H, W, C), (0, 3, 1, 2))
    return (loss.reshape(()), qst_out, perp.reshape(()), enc)
